# Initial kernel scaffold; baseline (speedup 1.0000x reference)
import jax
import jax.numpy as jnp
from jax.experimental import pallas as pl


def kernel(x, tok_emb, pos_emb):
    raise NotImplementedError("write your pallas kernel here")


# speedup vs baseline: 1.2361x; 1.2361x over previous
import math
import functools

import jax
import jax.numpy as jnp
from jax import lax
from jax.experimental import pallas as pl
from jax.experimental.pallas import tpu as pltpu
from jax.experimental.pallas import tpu_sc as plsc

NC = 2
NS = 16
NW = NC * NS
LANES = 16


def _make_kernel(BS, S, D, CH, interpret=False):
    RPW = BS // NW
    n_chunks = RPW // CH
    scale = 1.0 / math.sqrt(D)
    mesh = plsc.VectorSubcoreMesh(
        core_axis_name="c", subcore_axis_name="s",
        num_cores=NC, num_subcores=NS)

    @functools.partial(
        pl.kernel,
        out_type=jax.ShapeDtypeStruct((BS, D), jnp.float32),
        mesh=mesh,
        scratch_types=[
            pltpu.VMEM((RPW,), jnp.int32),
            pltpu.VMEM((CH, D), jnp.float32),
            pltpu.VMEM((CH, D), jnp.float32),
            pltpu.SemaphoreType.DMA,
            pltpu.SemaphoreType.DMA,
        ],
        interpret=interpret,
    )
    def k(x_hbm, tok_hbm, pos_hbm, out_hbm, idx_v, rows_v, pos_v, gsem, psem):
        wid = lax.axis_index("s") * NC + lax.axis_index("c")
        base = wid * RPW
        pos_base = base % S
        pltpu.sync_copy(x_hbm.at[pl.ds(base, RPW)], idx_v)

        @pl.loop(0, n_chunks)
        def _chunk(j):
            off = j * CH
            g = pltpu.async_copy(
                tok_hbm.at[idx_v.at[pl.ds(off, CH)]], rows_v, gsem)
            p = pltpu.async_copy(
                pos_hbm.at[pl.ds(pos_base + off, CH), :], pos_v, psem)
            g.wait()
            p.wait()

            @pl.loop(0, CH)
            def _row(r):
                @pl.loop(0, D // LANES, unroll=4)
                def _col(c):
                    cs = c * LANES
                    rows_v[r, pl.ds(cs, LANES)] = (
                        rows_v[r, pl.ds(cs, LANES)] * scale
                        + pos_v[r, pl.ds(cs, LANES)])

            pltpu.sync_copy(rows_v, out_hbm.at[pl.ds(base + off, CH), :])

    return k


@jax.jit
def kernel(x, tok_emb, pos_emb):
    B, S = x.shape
    D = tok_emb.shape[1]
    BS = B * S
    xf = x.reshape(BS).astype(jnp.int32)
    out = _make_kernel(BS, S, D, CH=32)(xf, tok_emb, pos_emb)
    return out.reshape(B, S, D)
